# baseline (speedup 1.0000x reference)
import functools

import jax
import jax.numpy as jnp
from jax import lax
from jax.experimental import pallas as pl
from jax.experimental.pallas import tpu as pltpu
from jax.experimental.pallas import tpu_sc as plsc

N_CTX = 8192
D_MODEL = 2048
BATCH = 4
NUM_WORKERS = 32
ROWS_PER_WORKER = N_CTX // NUM_WORKERS
_SIZES = [32, 24, 32, 24, 32, 24, 32, 24, 32]
_OFFS = [sum(_SIZES[:i]) for i in range(len(_SIZES))]
_NCH = len(_SIZES)


@functools.partial(
    pl.kernel,
    mesh=plsc.VectorSubcoreMesh(core_axis_name="c", subcore_axis_name="s"),
    out_type=jax.ShapeDtypeStruct((BATCH, N_CTX, D_MODEL), jnp.float32),
    scratch_types=[
        pltpu.VMEM((32, D_MODEL), jnp.float32),
        pltpu.VMEM((24, D_MODEL), jnp.float32),
        pltpu.SemaphoreType.DMA,
        pltpu.SemaphoreType.DMA,
        pltpu.SemaphoreType.DMA,
    ],
)
def _pos_broadcast(w_hbm, out_hbm, buf_a, buf_b, rsem, wsem_a, wsem_b):
    wid = lax.axis_index("s") * 2 + lax.axis_index("c")
    base = wid * ROWS_PER_WORKER
    bufs = (buf_a, buf_b)
    wsems = (wsem_a, wsem_b)

    def rd(c):
        return pltpu.async_copy(
            w_hbm.at[pl.ds(base + _OFFS[c], _SIZES[c])], bufs[c % 2], rsem)

    reads = [None] * _NCH
    writes = [None] * _NCH
    reads[0] = rd(0)
    for c in range(_NCH):
        reads[c].wait()
        if c + 1 < _NCH:
            if c >= 1:
                for w in writes[c - 1]:
                    w.wait()
            reads[c + 1] = rd(c + 1)
        writes[c] = [
            pltpu.async_copy(
                bufs[c % 2],
                out_hbm.at[b, pl.ds(base + _OFFS[c], _SIZES[c])],
                wsems[c % 2])
            for b in range(BATCH)
        ]
    for c in (_NCH - 2, _NCH - 1):
        for w in writes[c]:
            w.wait()


def kernel(tokens, W_pos):
    del tokens
    return _pos_broadcast(W_pos)


# speedup vs baseline: 1.0143x; 1.0143x over previous
import functools

import jax
import jax.numpy as jnp
from jax import lax
from jax.experimental import pallas as pl
from jax.experimental.pallas import tpu as pltpu
from jax.experimental.pallas import tpu_sc as plsc

N_CTX = 8192
D_MODEL = 2048
BATCH = 4
NUM_WORKERS = 32
ROWS_PER_WORKER = N_CTX // NUM_WORKERS
CHUNK = 56
_CHUNKS = [(i * CHUNK, CHUNK) for i in range(4)] + [(4 * CHUNK, 32)]


@functools.partial(
    pl.kernel,
    mesh=plsc.VectorSubcoreMesh(core_axis_name="c", subcore_axis_name="s"),
    out_type=jax.ShapeDtypeStruct((BATCH, N_CTX, D_MODEL), jnp.float32),
    scratch_types=[
        pltpu.VMEM((CHUNK, D_MODEL), jnp.float32),
        pltpu.SemaphoreType.DMA,
        pltpu.SemaphoreType.DMA,
    ],
)
def _pos_broadcast(w_hbm, out_hbm, buf, rsem, wsem):
    wid = lax.axis_index("s") * 2 + lax.axis_index("c")
    base = wid * ROWS_PER_WORKER

    for off, n in _CHUNKS:
        r0 = base + off
        pltpu.async_copy(w_hbm.at[pl.ds(r0, n)], buf.at[pl.ds(0, n)], rsem).wait()
        copies = [
            pltpu.async_copy(
                buf.at[pl.ds(0, n)], out_hbm.at[b, pl.ds(r0, n)], wsem)
            for b in range(BATCH)
        ]
        for cp in copies:
            cp.wait()


def kernel(tokens, W_pos):
    del tokens
    return _pos_broadcast(W_pos)
